# baseline (speedup 1.0000x reference)
import functools

import jax
import jax.numpy as jnp
from jax import lax
from jax.experimental import pallas as pl
from jax.experimental.pallas import tpu as pltpu
from jax.experimental.pallas import tpu_sc as plsc

N = 10000
E = 320000
D = 128

_CB = 256
_NCH = E // _CB
_NSUB = 16
_NIT = 80
_CA = 512
_NCA = E // _CA
_NITA = (_NCA + 31) // 32
_NPAD = 10240
_RPT = _NPAD // _NSUB


def _stage1_body(x_ref, wt_ref, xs_ref):
    xs_ref[...] = jnp.dot(x_ref[...], wt_ref[...],
                          preferred_element_type=jnp.float32)


def _stage1_call(x, wt):
    blk = 2000
    return pl.pallas_call(
        _stage1_body,
        grid=(N // blk,),
        in_specs=[
            pl.BlockSpec((blk, D), lambda i: (i, 0)),
            pl.BlockSpec((D, D), lambda i: (0, 0)),
        ],
        out_specs=pl.BlockSpec((blk, D), lambda i: (i, 0)),
        out_shape=jax.ShapeDtypeStruct((N, D), jnp.float32),
    )(x, wt)


def _ab_body(xs_ref, att_ref, bias_ref, ab_ref):
    ab = lax.dot_general(att_ref[...], xs_ref[...], (((1,), (1,)), ((), ())),
                         preferred_element_type=jnp.float32)
    ab_ref[...] = ab
    ab_ref[0:1, :] = ab[0:1, :] + bias_ref[0]


def _ab_call(xs, att2, bias):
    return pl.pallas_call(
        _ab_body,
        in_specs=[
            pl.BlockSpec((N, D), lambda: (0, 0)),
            pl.BlockSpec((2, D), lambda: (0, 0)),
            pl.BlockSpec(memory_space=pltpu.SMEM),
        ],
        out_specs=pl.BlockSpec((2, N), lambda: (0, 0)),
        out_shape=jax.ShapeDtypeStruct((2, N), jnp.float32),
    )(xs, att2, bias)


def _edge_scale_body(alpha_hbm, beta_hbm, src_hbm, dst_hbm, sc_out,
                     alpha_v, beta_v, isrc, idst, sbuf):
    c = lax.axis_index("c")
    s = lax.axis_index("s")
    w = s * 2 + c
    pltpu.sync_copy(alpha_hbm, alpha_v)
    pltpu.sync_copy(beta_hbm, beta_v)

    def body(i, carry):
        cid = w + 32 * i

        @pl.when(cid < _NCA)
        def _():
            base = cid * _CA
            pltpu.sync_copy(src_hbm.at[pl.ds(base, _CA)], isrc)
            pltpu.sync_copy(dst_hbm.at[pl.ds(base, _CA)], idst)
            for g in range(_CA // 16):
                dsts = idst[pl.ds(g * 16, 16)]
                srcs = isrc[pl.ds(g * 16, 16)]
                z = (plsc.load_gather(alpha_v, [dsts])
                     + plsc.load_gather(beta_v, [srcs]))
                sbuf[pl.ds(g * 16, 16)] = jnp.exp(jnp.maximum(z, 0.2 * z))
            pltpu.sync_copy(sbuf, sc_out.at[pl.ds(base, _CA)])
        return carry
    lax.fori_loop(0, _NITA, body, 0)


def _edge_scale_call(alpha, beta, srcs, dsts):
    mesh = plsc.VectorSubcoreMesh(core_axis_name="c", subcore_axis_name="s")
    f = pl.kernel(
        _edge_scale_body,
        out_type=jax.ShapeDtypeStruct((E,), jnp.float32),
        mesh=mesh,
        scratch_types=[
            pltpu.VMEM((N,), jnp.float32),
            pltpu.VMEM((N,), jnp.float32),
            pltpu.VMEM((_CA,), jnp.int32),
            pltpu.VMEM((_CA,), jnp.int32),
            pltpu.VMEM((_CA,), jnp.float32),
        ],
        compiler_params=pltpu.CompilerParams(needs_layout_passes=False),
    )
    return f(alpha, beta, srcs, dsts)


def _sc_body(xs_hbm, x_hbm, pk_hbm, zeros_hbm,
             out0, out1, out2, out3,
             acc, idxb0, idxb1, rows0, rows1, sem0, sem1,
             scat0, scat1, fsem0, fsem1):
    c = lax.axis_index("c")
    s = lax.axis_index("s")
    rbase = s * _RPT
    idxb = (idxb0, idxb1)
    rows = (rows0, rows1)
    sems = (sem0, sem1)
    scats = (scat0, scat1)
    fsems = (fsem0, fsem1)

    def fill_idx_start(slot, p, cid, fsem):
        @pl.when(cid < _NCH)
        def _():
            pltpu.async_copy(pk_hbm.at[p * _NCH + cid], idxb[slot], fsem)

    def fill_idx_wait(slot, p, cid, fsem):
        @pl.when(cid < _NCH)
        def _():
            pltpu.make_async_copy(pk_hbm.at[p * _NCH + cid], idxb[slot],
                                  fsem).wait()

    def start_half(slot, j, table, cid):
        @pl.when(cid < _NCH)
        def _():
            pltpu.async_copy(table.at[idxb[slot].at[j]], rows[j], sems[j])

    def scale_edges(slot, j):
        r16 = jnp.full((16,), 4 + j, dtype=jnp.int32)
        rw = rows[j]

        @plsc.parallel_loop(0, 128, step=1, unroll=32)
        def _(e):
            e16 = jnp.full((16,), e, dtype=jnp.int32)
            sv = plsc.bitcast(plsc.load_gather(idxb[slot], [r16, e16]),
                              jnp.float32)
            for q in range(8):
                rw[e, pl.ds(q * 16, 16)] = rw[e, pl.ds(q * 16, 16)] * sv

    def finish_half(slot, j, table, cid):
        @pl.when(cid < _NCH)
        def _():
            pltpu.make_async_copy(table.at[idxb[slot].at[j]], rows[j],
                                  sems[j]).wait()
            scale_edges(slot, j)
            pltpu.async_copy(rows[j], acc.at[idxb[slot].at[2 + j]], scats[j],
                             add=True)

    def wait_scat(slot, j, cid):
        @pl.when(cid < _NCH)
        def _():
            pltpu.make_async_copy(rows[j], acc.at[idxb[slot].at[2 + j]],
                                  scats[j]).wait()

    def run_pass(p, table):
        fill_idx_start(0, p, s, fsem0)
        fill_idx_wait(0, p, s, fsem0)
        start_half(0, 0, table, s)
        start_half(0, 1, table, s)
        fill_idx_start(1, p, s + _NSUB, fsem1)

        def body(i, carry):
            for uc in (0, 1):
                cid = s + _NSUB * (2 * i + uc)
                ncid = cid + _NSUB
                finish_half(uc, 0, table, cid)
                fill_idx_wait(1 - uc, p, ncid, fsems[1 - uc])
                finish_half(uc, 1, table, cid)
                wait_scat(uc, 0, cid)
                start_half(1 - uc, 0, table, ncid)
                wait_scat(uc, 1, cid)
                start_half(1 - uc, 1, table, ncid)
                fill_idx_start(uc, p, cid + 2 * _NSUB, fsems[uc])
            return carry
        lax.fori_loop(0, _NIT // 2, body, 0)

    for phase in range(2):
        pltpu.sync_copy(zeros_hbm, acc.at[pl.ds(rbase, _RPT)])
        plsc.subcore_barrier()

        @pl.when(c == 0)
        def _():
            if phase == 0:
                run_pass(0, xs_hbm)
            else:
                run_pass(1, x_hbm)

        @pl.when(c == 1)
        def _():
            run_pass(2 + phase, x_hbm)

        plsc.subcore_barrier()
        o_c0 = out0 if phase == 0 else out1
        o_c1 = out2 if phase == 0 else out3

        @pl.when(c == 0)
        def _():
            pltpu.sync_copy(acc.at[pl.ds(rbase, _RPT)], o_c0.at[pl.ds(rbase, _RPT)])

        @pl.when(c == 1)
        def _():
            pltpu.sync_copy(acc.at[pl.ds(rbase, _RPT)], o_c1.at[pl.ds(rbase, _RPT)])


def _sc_call(xs, x, pk, zeros):
    mesh = plsc.VectorSubcoreMesh(core_axis_name="c", subcore_axis_name="s")
    f = pl.kernel(
        _sc_body,
        out_type=[jax.ShapeDtypeStruct((_NPAD, D), jnp.float32)] * 4,
        mesh=mesh,
        scratch_types=[
            pltpu.VMEM_SHARED((_NPAD, D), jnp.float32),
            pltpu.VMEM((8, 128), jnp.int32),
            pltpu.VMEM((8, 128), jnp.int32),
            pltpu.VMEM((128, D), jnp.float32),
            pltpu.VMEM((128, D), jnp.float32),
            pltpu.SemaphoreType.DMA,
            pltpu.SemaphoreType.DMA,
            pltpu.SemaphoreType.DMA,
            pltpu.SemaphoreType.DMA,
            pltpu.SemaphoreType.DMA,
            pltpu.SemaphoreType.DMA,
        ],
        compiler_params=pltpu.CompilerParams(needs_layout_passes=False),
    )
    return f(xs, x, pk, zeros)


def _stage3_body(sh_ref, a0_ref, a1_ref, a2_ref, wlt_ref, aw_ref, out_ref):
    wlt = wlt_ref[...]
    aw = aw_ref[...]
    hs, ls = [], []
    for ar in (a0_ref, a1_ref, a2_ref):
        h = jnp.dot(ar[...], wlt, preferred_element_type=jnp.float32)
        h = jnp.maximum(h, 0.01 * h)
        hs.append(h)
        ls.append(jnp.sum(h * aw, axis=1, keepdims=True))
    m = jnp.maximum(jnp.maximum(ls[0], ls[1]), ls[2])
    es = [jnp.exp(l - m) for l in ls]
    z = es[0] + es[1] + es[2]
    out_ref[...] = sh_ref[...] + (es[0] * hs[0] + es[1] * hs[1] + es[2] * hs[2]) / z


def _stage3_call(short, a0, a1, a2, wlt, aw):
    blk = 2000
    return pl.pallas_call(
        _stage3_body,
        grid=(N // blk,),
        in_specs=[
            pl.BlockSpec((blk, D), lambda i: (i, 0)),
            pl.BlockSpec((blk, D), lambda i: (i, 0)),
            pl.BlockSpec((blk, D), lambda i: (i, 0)),
            pl.BlockSpec((blk, D), lambda i: (i, 0)),
            pl.BlockSpec((D, D), lambda i: (0, 0)),
            pl.BlockSpec((1, D), lambda i: (0, 0)),
        ],
        out_specs=pl.BlockSpec((blk, D), lambda i: (i, 0)),
        out_shape=jax.ShapeDtypeStruct((N, D), jnp.float32),
    )(short, a0, a1, a2, wlt, aw)


def kernel(x, edge_index, adj_edge_index, adj_values,
           W_s, att_s_w, att_s_b, W_l, att_l_w, att_l_b):
    att2 = jnp.reshape(att_s_w, (2, D))
    xs = _stage1_call(x, W_s.T)
    ab = _ab_call(xs, att2, att_s_b)
    zeros = jnp.zeros((_RPT, D), jnp.float32)

    def pack(srcs, dsts, valbits):
        s3 = srcs.reshape(_NCH, 2, 128)
        d3 = dsts.reshape(_NCH, 2, 128)
        v3 = valbits.reshape(_NCH, 2, 128)
        z3 = jnp.zeros_like(s3)
        return jnp.concatenate([s3, d3, v3, z3], axis=1)

    escale = _edge_scale_call(ab[0], ab[1], edge_index[0], edge_index[1])
    vbits = jax.lax.bitcast_convert_type(adj_values, jnp.int32)
    ebits = jax.lax.bitcast_convert_type(escale, jnp.int32)
    pk = jnp.concatenate([
        pack(edge_index[0], edge_index[1], ebits),
        pack(adj_edge_index[0, 1], adj_edge_index[0, 0], vbits[0]),
        pack(adj_edge_index[1, 1], adj_edge_index[1, 0], vbits[1]),
        pack(adj_edge_index[2, 1], adj_edge_index[2, 0], vbits[2]),
    ], axis=0)
    short, ax0, ax1, ax2 = _sc_call(xs, x, pk, zeros)
    return _stage3_call(short, ax0, ax1, ax2, W_l.T, att_l_w)


# speedup vs baseline: 1.0412x; 1.0412x over previous
import functools

import jax
import jax.numpy as jnp
from jax import lax
from jax.experimental import pallas as pl
from jax.experimental.pallas import tpu as pltpu
from jax.experimental.pallas import tpu_sc as plsc

N = 10000
E = 320000
D = 128

_CB = 256
_NCH = E // _CB
_NSUB = 16
_NIT = 80
_CA = 512
_NCA = E // _CA
_NITA = (_NCA + 31) // 32
_NPAD = 10240
_RPT = _NPAD // _NSUB


def _stage1_body(x_ref, wt_ref, xs_ref):
    xs_ref[...] = jnp.dot(x_ref[...], wt_ref[...],
                          preferred_element_type=jnp.float32)


def _stage1_call(x, wt):
    blk = 2000
    return pl.pallas_call(
        _stage1_body,
        grid=(N // blk,),
        in_specs=[
            pl.BlockSpec((blk, D), lambda i: (i, 0)),
            pl.BlockSpec((D, D), lambda i: (0, 0)),
        ],
        out_specs=pl.BlockSpec((blk, D), lambda i: (i, 0)),
        out_shape=jax.ShapeDtypeStruct((N, D), jnp.float32),
    )(x, wt)


def _ab_body(xs_ref, att_ref, bias_ref, ab_ref):
    ab = lax.dot_general(att_ref[...], xs_ref[...], (((1,), (1,)), ((), ())),
                         preferred_element_type=jnp.float32)
    ab_ref[...] = ab
    ab_ref[0:1, :] = ab[0:1, :] + bias_ref[0]


def _ab_call(xs, att2, bias):
    return pl.pallas_call(
        _ab_body,
        in_specs=[
            pl.BlockSpec((N, D), lambda: (0, 0)),
            pl.BlockSpec((2, D), lambda: (0, 0)),
            pl.BlockSpec(memory_space=pltpu.SMEM),
        ],
        out_specs=pl.BlockSpec((2, N), lambda: (0, 0)),
        out_shape=jax.ShapeDtypeStruct((2, N), jnp.float32),
    )(xs, att2, bias)


def _edge_scale_body(alpha_hbm, beta_hbm, src_hbm, dst_hbm, sc_out,
                     alpha_v, beta_v, isrc, idst, sbuf):
    c = lax.axis_index("c")
    s = lax.axis_index("s")
    w = s * 2 + c
    pltpu.sync_copy(alpha_hbm, alpha_v)
    pltpu.sync_copy(beta_hbm, beta_v)

    def body(i, carry):
        cid = w + 32 * i

        @pl.when(cid < _NCA)
        def _():
            base = cid * _CA
            pltpu.sync_copy(src_hbm.at[pl.ds(base, _CA)], isrc)
            pltpu.sync_copy(dst_hbm.at[pl.ds(base, _CA)], idst)
            for g in range(_CA // 16):
                dsts = idst[pl.ds(g * 16, 16)]
                srcs = isrc[pl.ds(g * 16, 16)]
                z = (plsc.load_gather(alpha_v, [dsts])
                     + plsc.load_gather(beta_v, [srcs]))
                sbuf[pl.ds(g * 16, 16)] = jnp.exp(jnp.maximum(z, 0.2 * z))
            pltpu.sync_copy(sbuf, sc_out.at[pl.ds(base, _CA)])
        return carry
    lax.fori_loop(0, _NITA, body, 0)


def _edge_scale_call(alpha, beta, srcs, dsts):
    mesh = plsc.VectorSubcoreMesh(core_axis_name="c", subcore_axis_name="s")
    f = pl.kernel(
        _edge_scale_body,
        out_type=jax.ShapeDtypeStruct((E,), jnp.float32),
        mesh=mesh,
        scratch_types=[
            pltpu.VMEM((N,), jnp.float32),
            pltpu.VMEM((N,), jnp.float32),
            pltpu.VMEM((_CA,), jnp.int32),
            pltpu.VMEM((_CA,), jnp.int32),
            pltpu.VMEM((_CA,), jnp.float32),
        ],
        compiler_params=pltpu.CompilerParams(needs_layout_passes=False),
    )
    return f(alpha, beta, srcs, dsts)


def _sc_body(xs_hbm, x_hbm, pk_hbm, es_hbm, zeros_hbm,
             out0, out1, out2, out3,
             acc, idxb0, idxb1, rows0, rows1, sb0, sb1, sem0, sem1,
             scat0, scat1, fsem0, fsem1):
    c = lax.axis_index("c")
    s = lax.axis_index("s")
    rbase = s * _RPT
    idxb = (idxb0, idxb1)
    rows = (rows0, rows1)
    sems = (sem0, sem1)
    scats = (scat0, scat1)
    fsems = (fsem0, fsem1)
    sb = (sb0, sb1)

    def fill_idx_start(slot, p, cid, fsem):
        @pl.when(cid < _NCH)
        def _():
            pltpu.async_copy(pk_hbm.at[p * _NCH + cid], idxb[slot], fsem)

    def fill_idx_wait(slot, p, cid, fsem):
        @pl.when(cid < _NCH)
        def _():
            pltpu.make_async_copy(pk_hbm.at[p * _NCH + cid], idxb[slot],
                                  fsem).wait()

    def start_half(slot, j, table, cid, short):
        @pl.when(cid < _NCH)
        def _():
            pltpu.async_copy(table.at[idxb[slot].at[j]], rows[j], sems[j])
            if short:
                pltpu.async_copy(es_hbm.at[pl.ds(cid * _CB + j * 128, 128)],
                                 sb[j], sems[j])

    def scale_edges(slot, j, short):
        r16 = jnp.full((16,), 4 + j, dtype=jnp.int32)
        rw = rows[j]

        @plsc.parallel_loop(0, 128, step=1, unroll=16)
        def _(e):
            e16 = jnp.full((16,), e, dtype=jnp.int32)
            if short:
                sv = plsc.load_gather(sb[j], [e16])
            else:
                sv = plsc.bitcast(plsc.load_gather(idxb[slot], [r16, e16]),
                                  jnp.float32)
            for q in range(8):
                rw[e, pl.ds(q * 16, 16)] = rw[e, pl.ds(q * 16, 16)] * sv

    def finish_half(slot, j, table, cid, short):
        @pl.when(cid < _NCH)
        def _():
            pltpu.make_async_copy(table.at[idxb[slot].at[j]], rows[j],
                                  sems[j]).wait()
            if short:
                pltpu.make_async_copy(
                    es_hbm.at[pl.ds(cid * _CB + j * 128, 128)], sb[j],
                    sems[j]).wait()
            scale_edges(slot, j, short)
            pltpu.async_copy(rows[j], acc.at[idxb[slot].at[2 + j]], scats[j],
                             add=True)

    def wait_scat(slot, j, cid):
        @pl.when(cid < _NCH)
        def _():
            pltpu.make_async_copy(rows[j], acc.at[idxb[slot].at[2 + j]],
                                  scats[j]).wait()

    def run_pass(p, table, short=False):
        fill_idx_start(0, p, s, fsem0)
        fill_idx_wait(0, p, s, fsem0)
        start_half(0, 0, table, s, short)
        start_half(0, 1, table, s, short)
        fill_idx_start(1, p, s + _NSUB, fsem1)

        def body(i, carry):
            for uc in (0, 1):
                cid = s + _NSUB * (2 * i + uc)
                ncid = cid + _NSUB
                finish_half(uc, 0, table, cid, short)
                fill_idx_wait(1 - uc, p, ncid, fsems[1 - uc])
                finish_half(uc, 1, table, cid, short)
                wait_scat(uc, 0, cid)
                start_half(1 - uc, 0, table, ncid, short)
                wait_scat(uc, 1, cid)
                start_half(1 - uc, 1, table, ncid, short)
                fill_idx_start(uc, p, cid + 2 * _NSUB, fsems[uc])
            return carry
        lax.fori_loop(0, _NIT // 2, body, 0)

    for phase in range(2):
        pltpu.sync_copy(zeros_hbm, acc.at[pl.ds(rbase, _RPT)])
        plsc.subcore_barrier()

        @pl.when(c == 0)
        def _():
            if phase == 0:
                run_pass(0, xs_hbm, short=True)
            else:
                run_pass(1, x_hbm)

        @pl.when(c == 1)
        def _():
            run_pass(2 + phase, x_hbm)

        plsc.subcore_barrier()
        o_c0 = out0 if phase == 0 else out1
        o_c1 = out2 if phase == 0 else out3

        @pl.when(c == 0)
        def _():
            pltpu.sync_copy(acc.at[pl.ds(rbase, _RPT)], o_c0.at[pl.ds(rbase, _RPT)])

        @pl.when(c == 1)
        def _():
            pltpu.sync_copy(acc.at[pl.ds(rbase, _RPT)], o_c1.at[pl.ds(rbase, _RPT)])


def _sc_call(xs, x, pk, escale, zeros):
    mesh = plsc.VectorSubcoreMesh(core_axis_name="c", subcore_axis_name="s")
    f = pl.kernel(
        _sc_body,
        out_type=[jax.ShapeDtypeStruct((_NPAD, D), jnp.float32)] * 4,
        mesh=mesh,
        scratch_types=[
            pltpu.VMEM_SHARED((_NPAD, D), jnp.float32),
            pltpu.VMEM((8, 128), jnp.int32),
            pltpu.VMEM((8, 128), jnp.int32),
            pltpu.VMEM((128, D), jnp.float32),
            pltpu.VMEM((128, D), jnp.float32),
            pltpu.VMEM((128,), jnp.float32),
            pltpu.VMEM((128,), jnp.float32),
            pltpu.SemaphoreType.DMA,
            pltpu.SemaphoreType.DMA,
            pltpu.SemaphoreType.DMA,
            pltpu.SemaphoreType.DMA,
            pltpu.SemaphoreType.DMA,
            pltpu.SemaphoreType.DMA,
        ],
        compiler_params=pltpu.CompilerParams(needs_layout_passes=False),
    )
    return f(xs, x, pk, escale, zeros)


def _stage3_body(sh_ref, a0_ref, a1_ref, a2_ref, wlt_ref, aw_ref, out_ref):
    wlt = wlt_ref[...]
    aw = aw_ref[...]
    hs, ls = [], []
    for ar in (a0_ref, a1_ref, a2_ref):
        h = jnp.dot(ar[...], wlt, preferred_element_type=jnp.float32)
        h = jnp.maximum(h, 0.01 * h)
        hs.append(h)
        ls.append(jnp.sum(h * aw, axis=1, keepdims=True))
    m = jnp.maximum(jnp.maximum(ls[0], ls[1]), ls[2])
    es = [jnp.exp(l - m) for l in ls]
    z = es[0] + es[1] + es[2]
    out_ref[...] = sh_ref[...] + (es[0] * hs[0] + es[1] * hs[1] + es[2] * hs[2]) / z


def _stage3_call(short, a0, a1, a2, wlt, aw):
    blk = 2000
    return pl.pallas_call(
        _stage3_body,
        grid=(N // blk,),
        in_specs=[
            pl.BlockSpec((blk, D), lambda i: (i, 0)),
            pl.BlockSpec((blk, D), lambda i: (i, 0)),
            pl.BlockSpec((blk, D), lambda i: (i, 0)),
            pl.BlockSpec((blk, D), lambda i: (i, 0)),
            pl.BlockSpec((D, D), lambda i: (0, 0)),
            pl.BlockSpec((1, D), lambda i: (0, 0)),
        ],
        out_specs=pl.BlockSpec((blk, D), lambda i: (i, 0)),
        out_shape=jax.ShapeDtypeStruct((N, D), jnp.float32),
    )(short, a0, a1, a2, wlt, aw)


def kernel(x, edge_index, adj_edge_index, adj_values,
           W_s, att_s_w, att_s_b, W_l, att_l_w, att_l_b):
    att2 = jnp.reshape(att_s_w, (2, D))
    xs = _stage1_call(x, W_s.T)
    ab = _ab_call(xs, att2, att_s_b)
    zeros = jnp.zeros((_RPT, D), jnp.float32)

    def pack(srcs, dsts, valbits):
        s3 = srcs.reshape(_NCH, 2, 128)
        d3 = dsts.reshape(_NCH, 2, 128)
        v3 = valbits.reshape(_NCH, 2, 128)
        z3 = jnp.zeros_like(s3)
        return jnp.concatenate([s3, d3, v3, z3], axis=1)

    escale = _edge_scale_call(ab[0], ab[1], edge_index[0], edge_index[1])
    vbits = jax.lax.bitcast_convert_type(adj_values, jnp.int32)
    zeroe = jnp.zeros((E,), jnp.int32)
    pk = jnp.concatenate([
        pack(edge_index[0], edge_index[1], zeroe),
        pack(adj_edge_index[0, 1], adj_edge_index[0, 0], vbits[0]),
        pack(adj_edge_index[1, 1], adj_edge_index[1, 0], vbits[1]),
        pack(adj_edge_index[2, 1], adj_edge_index[2, 0], vbits[2]),
    ], axis=0)
    short, ax0, ax1, ax2 = _sc_call(xs, x, pk, escale, zeros)
    return _stage3_call(short, ax0, ax1, ax2, W_l.T, att_l_w)


# speedup vs baseline: 1.0440x; 1.0027x over previous
import functools

import jax
import jax.numpy as jnp
from jax import lax
from jax.experimental import pallas as pl
from jax.experimental.pallas import tpu as pltpu
from jax.experimental.pallas import tpu_sc as plsc

N = 10000
E = 320000
D = 128

_CB = 256
_NCH = E // _CB
_NSUB = 16
_NIT = 80
_CA = 512
_NCA = E // _CA
_NITA = (_NCA + 31) // 32
_NPAD = 10240
_RPT = _NPAD // _NSUB


def _stage1_body(x_ref, wt_ref, xs_ref):
    xs_ref[...] = jnp.dot(x_ref[...], wt_ref[...],
                          preferred_element_type=jnp.float32)


def _stage1_call(x, wt):
    blk = 2000
    return pl.pallas_call(
        _stage1_body,
        grid=(N // blk,),
        in_specs=[
            pl.BlockSpec((blk, D), lambda i: (i, 0)),
            pl.BlockSpec((D, D), lambda i: (0, 0)),
        ],
        out_specs=pl.BlockSpec((blk, D), lambda i: (i, 0)),
        out_shape=jax.ShapeDtypeStruct((N, D), jnp.float32),
    )(x, wt)


def _ab_body(xs_ref, att_ref, bias_ref, ab_ref):
    ab = lax.dot_general(att_ref[...], xs_ref[...], (((1,), (1,)), ((), ())),
                         preferred_element_type=jnp.float32)
    ab_ref[...] = ab
    ab_ref[0:1, :] = ab[0:1, :] + bias_ref[0]


def _ab_call(xs, att2, bias):
    return pl.pallas_call(
        _ab_body,
        in_specs=[
            pl.BlockSpec((N, D), lambda: (0, 0)),
            pl.BlockSpec((2, D), lambda: (0, 0)),
            pl.BlockSpec(memory_space=pltpu.SMEM),
        ],
        out_specs=pl.BlockSpec((2, N), lambda: (0, 0)),
        out_shape=jax.ShapeDtypeStruct((2, N), jnp.float32),
    )(xs, att2, bias)


def _edge_scale_body(alpha_hbm, beta_hbm, src_hbm, dst_hbm, sc_out,
                     alpha_v, beta_v, isrc0, isrc1, idst0, idst1,
                     sbuf0, sbuf1, ds0, ds1, os0, os1):
    c = lax.axis_index("c")
    s = lax.axis_index("s")
    w = s * 2 + c
    isrc = (isrc0, isrc1)
    idst = (idst0, idst1)
    sbuf = (sbuf0, sbuf1)
    dsem = (ds0, ds1)
    osem = (os0, os1)
    pltpu.sync_copy(alpha_hbm, alpha_v)
    pltpu.sync_copy(beta_hbm, beta_v)

    def fetch(slot, cid):
        @pl.when(cid < _NCA)
        def _():
            base = cid * _CA
            pltpu.async_copy(src_hbm.at[pl.ds(base, _CA)], isrc[slot], dsem[slot])
            pltpu.async_copy(dst_hbm.at[pl.ds(base, _CA)], idst[slot], dsem[slot])

    def process(slot, cid):
        @pl.when(cid < _NCA)
        def _():
            base = cid * _CA
            pltpu.make_async_copy(src_hbm.at[pl.ds(base, _CA)], isrc[slot],
                                  dsem[slot]).wait()
            pltpu.make_async_copy(dst_hbm.at[pl.ds(base, _CA)], idst[slot],
                                  dsem[slot]).wait()
            for g in range(_CA // 16):
                dsts = idst[slot][pl.ds(g * 16, 16)]
                srcs = isrc[slot][pl.ds(g * 16, 16)]
                z = (plsc.load_gather(alpha_v, [dsts])
                     + plsc.load_gather(beta_v, [srcs]))
                sbuf[slot][pl.ds(g * 16, 16)] = jnp.exp(jnp.maximum(z, 0.2 * z))
            pltpu.async_copy(sbuf[slot], sc_out.at[pl.ds(base, _CA)], osem[slot])

    def drain(slot, cid):
        @pl.when((cid >= 0) & (cid < _NCA))
        def _():
            base = cid * _CA
            pltpu.make_async_copy(sbuf[slot], sc_out.at[pl.ds(base, _CA)],
                                  osem[slot]).wait()

    fetch(0, w)

    def body(i, carry):
        for u in (0, 1):
            cid = w + 32 * (2 * i + u)
            fetch(1 - u, cid + 32)
            drain(u, cid - 64)
            process(u, cid)
        return carry
    lax.fori_loop(0, _NITA // 2, body, 0)
    drain(0, w + 32 * (_NITA - 2))
    drain(1, w + 32 * (_NITA - 1))


def _edge_scale_call(alpha, beta, srcs, dsts):
    mesh = plsc.VectorSubcoreMesh(core_axis_name="c", subcore_axis_name="s")
    f = pl.kernel(
        _edge_scale_body,
        out_type=jax.ShapeDtypeStruct((E,), jnp.float32),
        mesh=mesh,
        scratch_types=[
            pltpu.VMEM((N,), jnp.float32),
            pltpu.VMEM((N,), jnp.float32),
            pltpu.VMEM((_CA,), jnp.int32),
            pltpu.VMEM((_CA,), jnp.int32),
            pltpu.VMEM((_CA,), jnp.int32),
            pltpu.VMEM((_CA,), jnp.int32),
            pltpu.VMEM((_CA,), jnp.float32),
            pltpu.VMEM((_CA,), jnp.float32),
            pltpu.SemaphoreType.DMA,
            pltpu.SemaphoreType.DMA,
            pltpu.SemaphoreType.DMA,
            pltpu.SemaphoreType.DMA,
        ],
        compiler_params=pltpu.CompilerParams(needs_layout_passes=False),
    )
    return f(alpha, beta, srcs, dsts)


def _sc_body(xs_hbm, x_hbm, pk_hbm, es_hbm, zeros_hbm,
             out0, out1, out2, out3,
             acc, idxb0, idxb1, rows0, rows1, sb0, sb1, sem0, sem1,
             scat0, scat1, fsem0, fsem1):
    c = lax.axis_index("c")
    s = lax.axis_index("s")
    rbase = s * _RPT
    idxb = (idxb0, idxb1)
    rows = (rows0, rows1)
    sems = (sem0, sem1)
    scats = (scat0, scat1)
    fsems = (fsem0, fsem1)
    sb = (sb0, sb1)

    def fill_idx_start(slot, p, cid, fsem):
        @pl.when(cid < _NCH)
        def _():
            pltpu.async_copy(pk_hbm.at[p * _NCH + cid], idxb[slot], fsem)

    def fill_idx_wait(slot, p, cid, fsem):
        @pl.when(cid < _NCH)
        def _():
            pltpu.make_async_copy(pk_hbm.at[p * _NCH + cid], idxb[slot],
                                  fsem).wait()

    def start_half(slot, j, table, cid, short):
        @pl.when(cid < _NCH)
        def _():
            pltpu.async_copy(table.at[idxb[slot].at[j]], rows[j], sems[j])
            if short:
                pltpu.async_copy(es_hbm.at[pl.ds(cid * _CB + j * 128, 128)],
                                 sb[j], sems[j])

    def scale_edges(slot, j, short):
        r16 = jnp.full((16,), 4 + j, dtype=jnp.int32)
        rw = rows[j]

        @plsc.parallel_loop(0, 128, step=1, unroll=16)
        def _(e):
            e16 = jnp.full((16,), e, dtype=jnp.int32)
            if short:
                sv = plsc.load_gather(sb[j], [e16])
            else:
                sv = plsc.bitcast(plsc.load_gather(idxb[slot], [r16, e16]),
                                  jnp.float32)
            for q in range(8):
                rw[e, pl.ds(q * 16, 16)] = rw[e, pl.ds(q * 16, 16)] * sv

    def finish_half(slot, j, table, cid, short):
        @pl.when(cid < _NCH)
        def _():
            pltpu.make_async_copy(table.at[idxb[slot].at[j]], rows[j],
                                  sems[j]).wait()
            if short:
                pltpu.make_async_copy(
                    es_hbm.at[pl.ds(cid * _CB + j * 128, 128)], sb[j],
                    sems[j]).wait()
            scale_edges(slot, j, short)
            pltpu.async_copy(rows[j], acc.at[idxb[slot].at[2 + j]], scats[j],
                             add=True)

    def wait_scat(slot, j, cid):
        @pl.when(cid < _NCH)
        def _():
            pltpu.make_async_copy(rows[j], acc.at[idxb[slot].at[2 + j]],
                                  scats[j]).wait()

    def run_pass(p, table, short=False):
        fill_idx_start(0, p, s, fsem0)
        fill_idx_wait(0, p, s, fsem0)
        start_half(0, 0, table, s, short)
        start_half(0, 1, table, s, short)
        fill_idx_start(1, p, s + _NSUB, fsem1)

        def body(i, carry):
            for uc in (0, 1):
                cid = s + _NSUB * (2 * i + uc)
                ncid = cid + _NSUB
                finish_half(uc, 0, table, cid, short)
                fill_idx_wait(1 - uc, p, ncid, fsems[1 - uc])
                finish_half(uc, 1, table, cid, short)
                wait_scat(uc, 0, cid)
                start_half(1 - uc, 0, table, ncid, short)
                wait_scat(uc, 1, cid)
                start_half(1 - uc, 1, table, ncid, short)
                fill_idx_start(uc, p, cid + 2 * _NSUB, fsems[uc])
            return carry
        lax.fori_loop(0, _NIT // 2, body, 0)

    for phase in range(2):
        pltpu.sync_copy(zeros_hbm, acc.at[pl.ds(rbase, _RPT)])
        plsc.subcore_barrier()

        @pl.when(c == 0)
        def _():
            if phase == 0:
                run_pass(0, xs_hbm, short=True)
            else:
                run_pass(1, x_hbm)

        @pl.when(c == 1)
        def _():
            run_pass(2 + phase, x_hbm)

        plsc.subcore_barrier()
        o_c0 = out0 if phase == 0 else out1
        o_c1 = out2 if phase == 0 else out3

        @pl.when(c == 0)
        def _():
            pltpu.sync_copy(acc.at[pl.ds(rbase, _RPT)], o_c0.at[pl.ds(rbase, _RPT)])

        @pl.when(c == 1)
        def _():
            pltpu.sync_copy(acc.at[pl.ds(rbase, _RPT)], o_c1.at[pl.ds(rbase, _RPT)])


def _sc_call(xs, x, pk, escale, zeros):
    mesh = plsc.VectorSubcoreMesh(core_axis_name="c", subcore_axis_name="s")
    f = pl.kernel(
        _sc_body,
        out_type=[jax.ShapeDtypeStruct((_NPAD, D), jnp.float32)] * 4,
        mesh=mesh,
        scratch_types=[
            pltpu.VMEM_SHARED((_NPAD, D), jnp.float32),
            pltpu.VMEM((8, 128), jnp.int32),
            pltpu.VMEM((8, 128), jnp.int32),
            pltpu.VMEM((128, D), jnp.float32),
            pltpu.VMEM((128, D), jnp.float32),
            pltpu.VMEM((128,), jnp.float32),
            pltpu.VMEM((128,), jnp.float32),
            pltpu.SemaphoreType.DMA,
            pltpu.SemaphoreType.DMA,
            pltpu.SemaphoreType.DMA,
            pltpu.SemaphoreType.DMA,
            pltpu.SemaphoreType.DMA,
            pltpu.SemaphoreType.DMA,
        ],
        compiler_params=pltpu.CompilerParams(needs_layout_passes=False),
    )
    return f(xs, x, pk, escale, zeros)


def _stage3_body(sh_ref, a0_ref, a1_ref, a2_ref, wlt_ref, aw_ref, out_ref):
    wlt = wlt_ref[...]
    aw = aw_ref[...]
    hs, ls = [], []
    for ar in (a0_ref, a1_ref, a2_ref):
        h = jnp.dot(ar[...], wlt, preferred_element_type=jnp.float32)
        h = jnp.maximum(h, 0.01 * h)
        hs.append(h)
        ls.append(jnp.sum(h * aw, axis=1, keepdims=True))
    m = jnp.maximum(jnp.maximum(ls[0], ls[1]), ls[2])
    es = [jnp.exp(l - m) for l in ls]
    z = es[0] + es[1] + es[2]
    out_ref[...] = sh_ref[...] + (es[0] * hs[0] + es[1] * hs[1] + es[2] * hs[2]) / z


def _stage3_call(short, a0, a1, a2, wlt, aw):
    blk = 2000
    return pl.pallas_call(
        _stage3_body,
        grid=(N // blk,),
        in_specs=[
            pl.BlockSpec((blk, D), lambda i: (i, 0)),
            pl.BlockSpec((blk, D), lambda i: (i, 0)),
            pl.BlockSpec((blk, D), lambda i: (i, 0)),
            pl.BlockSpec((blk, D), lambda i: (i, 0)),
            pl.BlockSpec((D, D), lambda i: (0, 0)),
            pl.BlockSpec((1, D), lambda i: (0, 0)),
        ],
        out_specs=pl.BlockSpec((blk, D), lambda i: (i, 0)),
        out_shape=jax.ShapeDtypeStruct((N, D), jnp.float32),
    )(short, a0, a1, a2, wlt, aw)


def kernel(x, edge_index, adj_edge_index, adj_values,
           W_s, att_s_w, att_s_b, W_l, att_l_w, att_l_b):
    att2 = jnp.reshape(att_s_w, (2, D))
    xs = _stage1_call(x, W_s.T)
    ab = _ab_call(xs, att2, att_s_b)
    zeros = jnp.zeros((_RPT, D), jnp.float32)

    def pack(srcs, dsts, valbits):
        s3 = srcs.reshape(_NCH, 2, 128)
        d3 = dsts.reshape(_NCH, 2, 128)
        v3 = valbits.reshape(_NCH, 2, 128)
        z3 = jnp.zeros_like(s3)
        return jnp.concatenate([s3, d3, v3, z3], axis=1)

    escale = _edge_scale_call(ab[0], ab[1], edge_index[0], edge_index[1])
    vbits = jax.lax.bitcast_convert_type(adj_values, jnp.int32)
    zeroe = jnp.zeros((E,), jnp.int32)
    pk = jnp.concatenate([
        pack(edge_index[0], edge_index[1], zeroe),
        pack(adj_edge_index[0, 1], adj_edge_index[0, 0], vbits[0]),
        pack(adj_edge_index[1, 1], adj_edge_index[1, 0], vbits[1]),
        pack(adj_edge_index[2, 1], adj_edge_index[2, 0], vbits[2]),
    ], axis=0)
    short, ax0, ax1, ax2 = _sc_call(xs, x, pk, escale, zeros)
    return _stage3_call(short, ax0, ax1, ax2, W_l.T, att_l_w)
